# baseline (speedup 1.0000x reference)
import functools

import jax
import jax.numpy as jnp
from jax.experimental import pallas as pl
from jax.experimental.pallas import tpu as pltpu
from jax.experimental.pallas import tpu_sc as plsc

E = 8
D = 768
H = 2048
T = 2048
B = 256
NB = T // B + E
PAD_T = NB * B
CH = 128
D2 = D // 2
NW = 32
RPW = PAD_T // NW
SD = 1024


def _route_body(eid_ref, dest_ref, be_ref, srt_ref):
    eid = eid_ref[...]
    lane = jax.lax.broadcasted_iota(jnp.int32, (T, E), 1)
    onehot = (lane == eid).astype(jnp.float32)

    counts = jnp.sum(onehot, axis=0, keepdims=True)
    padded = jnp.floor((counts + (B - 1)) * (1.0 / B)) * B
    upper = (jax.lax.broadcasted_iota(jnp.int32, (E, E), 0)
             < jax.lax.broadcasted_iota(jnp.int32, (E, E), 1))
    excl = jax.lax.dot_general(
        padded, upper.astype(jnp.float32), (((1,), (0,)), ((), ())))

    bstart = excl * (1.0 / B)
    bi = jax.lax.broadcasted_iota(jnp.int32, (NB, E), 0).astype(jnp.float32)
    be = jnp.sum((bi >= bstart).astype(jnp.float32), axis=1, keepdims=True) - 1.0
    be_ref[...] = be.astype(jnp.int32)

    ltri = (jax.lax.broadcasted_iota(jnp.int32, (CH, CH), 0)
            > jax.lax.broadcasted_iota(jnp.int32, (CH, CH), 1)).astype(jnp.float32)
    carry = jnp.zeros((1, E), jnp.float32)
    chunks = []
    for c in range(T // CH):
        oh = onehot[c * CH:(c + 1) * CH]
        prefix = jax.lax.dot_general(ltri, oh, (((1,), (0,)), ((), ())))
        d = jnp.sum(oh * (prefix + carry + excl), axis=1, keepdims=True)
        chunks.append(d.astype(jnp.int32))
        carry = carry + jnp.sum(oh, axis=0, keepdims=True)
    destv = jnp.concatenate(chunks, axis=0)
    dest_ref[...] = destv

    ids_row = jax.lax.broadcasted_iota(jnp.int32, (1, T), 1).astype(jnp.float32)
    hi_row = jnp.floor(ids_row * (1.0 / 16.0))
    lo_row = (ids_row - 16.0 * hi_row).astype(jnp.bfloat16)
    hi_row = hi_row.astype(jnp.bfloat16)
    for c4 in range(PAD_T // SD):
        dio = jax.lax.broadcasted_iota(jnp.int32, (T, SD), 1) + c4 * SD
        hit = (destv == dio).astype(jnp.bfloat16)
        s_hi = jax.lax.dot_general(hi_row, hit, (((1,), (0,)), ((), ())),
                                   preferred_element_type=jnp.float32)
        s_lo = jax.lax.dot_general(lo_row, hit, (((1,), (0,)), ((), ())),
                                   preferred_element_type=jnp.float32)
        srt_ref[c4:c4 + 1, :] = (16.0 * s_hi + s_lo).astype(jnp.int32)


_route = pl.pallas_call(
    _route_body,
    out_shape=(jax.ShapeDtypeStruct((T, 1), jnp.int32),
               jax.ShapeDtypeStruct((NB, 1), jnp.int32),
               jax.ShapeDtypeStruct((PAD_T // SD, SD), jnp.int32)),
)


def _ffn_body(be_ref, dest_ref, xs_ref, w1_ref, w3_ref, w2_ref, out_ref):
    del be_ref
    i = pl.program_id(0)

    @pl.when(i == 0)
    def _init():
        out_ref[...] = jnp.zeros((T, D), jnp.float32)

    xv = xs_ref[...]
    xv = jnp.clip(jnp.where(jnp.isnan(xv), 0.0, xv), -1e4, 1e4)
    xb = xv.astype(jnp.bfloat16)
    w1 = w1_ref[0].astype(jnp.bfloat16)
    w3 = w3_ref[0].astype(jnp.bfloat16)
    w2 = w2_ref[0].astype(jnp.bfloat16)
    a = jax.lax.dot_general(xb, w1, (((1,), (1,)), ((), ())),
                            preferred_element_type=jnp.float32)
    g = jax.lax.dot_general(xb, w3, (((1,), (1,)), ((), ())),
                            preferred_element_type=jnp.float32)
    h = (a * jax.nn.sigmoid(a) * g).astype(jnp.bfloat16)
    out_i = jax.lax.dot_general(
        h, w2, (((1,), (1,)), ((), ())),
        preferred_element_type=jnp.float32).astype(jnp.bfloat16)
    sel = (dest_ref[...] ==
           (jax.lax.broadcasted_iota(jnp.int32, (T, B), 1) + i * B))
    out_ref[...] += jax.lax.dot_general(
        sel.astype(jnp.bfloat16), out_i, (((1,), (0,)), ((), ())),
        preferred_element_type=jnp.float32)


_ffn = pl.pallas_call(
    _ffn_body,
    grid_spec=pltpu.PrefetchScalarGridSpec(
        num_scalar_prefetch=1,
        grid=(NB,),
        in_specs=[
            pl.BlockSpec((T, 1), lambda i, be: (0, 0)),
            pl.BlockSpec((B, D), lambda i, be: (i, 0)),
            pl.BlockSpec((1, H, D), lambda i, be: (be[i], 0, 0)),
            pl.BlockSpec((1, H, D), lambda i, be: (be[i], 0, 0)),
            pl.BlockSpec((1, D, H), lambda i, be: (be[i], 0, 0)),
        ],
        out_specs=pl.BlockSpec((T, D), lambda i, be: (0, 0)),
    ),
    out_shape=jax.ShapeDtypeStruct((T, D), jnp.float32),
)


@functools.lru_cache(maxsize=1)
def _sc_kernels():
    mesh = plsc.VectorSubcoreMesh(core_axis_name="c", subcore_axis_name="s")
    sc_scratch = [
        pltpu.VMEM((RPW,), jnp.int32),
        pltpu.VMEM((RPW, D), jnp.float32),
        pltpu.SemaphoreType.DMA,
    ]

    @functools.partial(
        pl.kernel,
        mesh=mesh,
        out_type=jax.ShapeDtypeStruct((PAD_T, D), jnp.float32),
        scratch_types=sc_scratch,
    )
    def dispatch(x_hbm, srt_hbm, out_hbm, idx_v, rows_v, sem):
        wid = jax.lax.axis_index("s") * 2 + jax.lax.axis_index("c")
        base = wid * RPW
        pltpu.sync_copy(srt_hbm.at[pl.ds(base, RPW)], idx_v)
        pltpu.async_copy(x_hbm.at[idx_v], rows_v, sem).wait()
        pltpu.sync_copy(rows_v, out_hbm.at[pl.ds(base, RPW)])

    return dispatch


def kernel(x, Wg, w1, w2, w3):
    orig_shape = x.shape
    xf = x.reshape(-1, x.shape[-1])
    scores = xf @ Wg.T
    _, expert_indices = jax.lax.top_k(scores, 1)
    dest2, be2, srt2 = _route(expert_indices)
    be = be2.reshape(NB)
    srt = srt2.reshape(PAD_T)
    dispatch = _sc_kernels()
    xs = dispatch(xf, srt)
    y = _ffn(be, dest2, xs, w1, w3, w2)
    return y.reshape(orig_shape)


# speedup vs baseline: 1.3623x; 1.3623x over previous
import functools

import jax
import jax.numpy as jnp
from jax.experimental import pallas as pl
from jax.experimental.pallas import tpu as pltpu
from jax.experimental.pallas import tpu_sc as plsc

E = 8
D = 768
H = 2048
T = 2048
B = 256
NB = T // B + E
PAD_T = NB * B
CH = 128
NW = 32
RPW = T // NW


def _route_body(eid_ref, dest_ref, be_ref):
    eid = eid_ref[...]
    lane = jax.lax.broadcasted_iota(jnp.int32, (T, E), 1)
    onehot = (lane == eid).astype(jnp.float32)

    counts = jnp.sum(onehot, axis=0, keepdims=True)
    padded = jnp.floor((counts + (B - 1)) * (1.0 / B)) * B
    upper = (jax.lax.broadcasted_iota(jnp.int32, (E, E), 0)
             < jax.lax.broadcasted_iota(jnp.int32, (E, E), 1))
    excl = jax.lax.dot_general(
        padded, upper.astype(jnp.float32), (((1,), (0,)), ((), ())))

    bstart = excl * (1.0 / B)
    bi = jax.lax.broadcasted_iota(jnp.int32, (NB, E), 0).astype(jnp.float32)
    be = jnp.sum((bi >= bstart).astype(jnp.float32), axis=1, keepdims=True) - 1.0
    be_ref[...] = be.astype(jnp.int32)

    ltri = (jax.lax.broadcasted_iota(jnp.int32, (CH, CH), 0)
            > jax.lax.broadcasted_iota(jnp.int32, (CH, CH), 1)).astype(jnp.float32)
    carry = jnp.zeros((1, E), jnp.float32)
    for c in range(T // CH):
        oh = onehot[c * CH:(c + 1) * CH]
        prefix = jax.lax.dot_general(ltri, oh, (((1,), (0,)), ((), ())))
        d = jnp.sum(oh * (prefix + carry + excl), axis=1, keepdims=True)
        dest_ref[c * CH:(c + 1) * CH, :] = d.astype(jnp.int32)
        carry = carry + jnp.sum(oh, axis=0, keepdims=True)


_route = pl.pallas_call(
    _route_body,
    out_shape=(jax.ShapeDtypeStruct((T, 1), jnp.int32),
               jax.ShapeDtypeStruct((NB, 1), jnp.int32)),
)


def _ffn_body(be_ref, xs_ref, w1_ref, w3_ref, w2_ref, out_ref):
    del be_ref
    xv = xs_ref[...]
    xv = jnp.clip(jnp.where(jnp.isnan(xv), 0.0, xv), -1e4, 1e4)
    xb = xv.astype(jnp.bfloat16)
    w1 = w1_ref[0].astype(jnp.bfloat16)
    w3 = w3_ref[0].astype(jnp.bfloat16)
    w2 = w2_ref[0].astype(jnp.bfloat16)
    a = jax.lax.dot_general(xb, w1, (((1,), (1,)), ((), ())),
                            preferred_element_type=jnp.float32)
    g = jax.lax.dot_general(xb, w3, (((1,), (1,)), ((), ())),
                            preferred_element_type=jnp.float32)
    h = (a * jax.nn.sigmoid(a) * g).astype(jnp.bfloat16)
    out_ref[...] = jax.lax.dot_general(
        h, w2, (((1,), (1,)), ((), ())),
        preferred_element_type=jnp.float32).astype(jnp.bfloat16)


_ffn = pl.pallas_call(
    _ffn_body,
    grid_spec=pltpu.PrefetchScalarGridSpec(
        num_scalar_prefetch=1,
        grid=(NB,),
        in_specs=[
            pl.BlockSpec((B, D), lambda i, be: (i, 0)),
            pl.BlockSpec((1, H, D), lambda i, be: (be[i], 0, 0)),
            pl.BlockSpec((1, H, D), lambda i, be: (be[i], 0, 0)),
            pl.BlockSpec((1, D, H), lambda i, be: (be[i], 0, 0)),
        ],
        out_specs=pl.BlockSpec((B, D), lambda i, be: (i, 0)),
    ),
    out_shape=jax.ShapeDtypeStruct((PAD_T, D), jnp.bfloat16),
)


def _combine_body(dest_ref, ys_ref, out_ref):
    dvals = dest_ref[...]
    sel = (dvals == jax.lax.broadcasted_iota(jnp.int32, (B, PAD_T), 1))
    out_ref[...] = jax.lax.dot_general(
        sel.astype(jnp.bfloat16), ys_ref[...], (((1,), (0,)), ((), ())),
        preferred_element_type=jnp.float32)


_combine = pl.pallas_call(
    _combine_body,
    grid=(T // B,),
    in_specs=[
        pl.BlockSpec((B, 1), lambda j: (j, 0)),
        pl.BlockSpec((PAD_T, D), lambda j: (0, 0)),
    ],
    out_specs=pl.BlockSpec((B, D), lambda j: (j, 0)),
    out_shape=jax.ShapeDtypeStruct((T, D), jnp.float32),
)


@functools.lru_cache(maxsize=1)
def _sc_kernels():
    mesh = plsc.VectorSubcoreMesh(core_axis_name="c", subcore_axis_name="s")
    sc_scratch = [
        pltpu.VMEM((RPW,), jnp.int32),
        pltpu.VMEM((RPW, D), jnp.float32),
        pltpu.SemaphoreType.DMA,
    ]

    @functools.partial(
        pl.kernel,
        mesh=mesh,
        out_type=jax.ShapeDtypeStruct((PAD_T, D), jnp.float32),
        scratch_types=sc_scratch,
    )
    def dispatch(x_hbm, dest_hbm, out_hbm, idx_v, rows_v, sem):
        wid = jax.lax.axis_index("s") * 2 + jax.lax.axis_index("c")
        base = wid * RPW
        pltpu.sync_copy(dest_hbm.at[pl.ds(base, RPW)], idx_v)
        pltpu.sync_copy(x_hbm.at[pl.ds(base, RPW)], rows_v)
        pltpu.async_copy(rows_v, out_hbm.at[idx_v], sem).wait()

    return dispatch


def kernel(x, Wg, w1, w2, w3):
    orig_shape = x.shape
    xf = x.reshape(-1, x.shape[-1])
    scores = xf @ Wg.T
    _, expert_indices = jax.lax.top_k(scores, 1)
    dest2, be2 = _route(expert_indices)
    dest = dest2.reshape(T)
    be = be2.reshape(NB)
    dispatch = _sc_kernels()
    xs = dispatch(xf, dest)
    ys = _ffn(be, xs, w1, w3, w2)
    y = _combine(dest2, ys)
    return y.reshape(orig_shape)


# speedup vs baseline: 1.3749x; 1.0093x over previous
import functools

import jax
import jax.numpy as jnp
from jax.experimental import pallas as pl
from jax.experimental.pallas import tpu as pltpu
from jax.experimental.pallas import tpu_sc as plsc

E = 8
D = 768
H = 2048
T = 2048
B = 256
NB = T // B + E
PAD_T = NB * B
CH = 128
NW = 32
RPW = T // NW
RPW2 = PAD_T // NW
SD = 1024


def _route_body(eid_ref, dest_ref, be_ref, srt_ref):
    eid = eid_ref[...]
    lane = jax.lax.broadcasted_iota(jnp.int32, (T, E), 1)
    onehot = (lane == eid).astype(jnp.float32)

    counts = jnp.sum(onehot, axis=0, keepdims=True)
    padded = jnp.floor((counts + (B - 1)) * (1.0 / B)) * B
    upper = (jax.lax.broadcasted_iota(jnp.int32, (E, E), 0)
             < jax.lax.broadcasted_iota(jnp.int32, (E, E), 1))
    excl = jax.lax.dot_general(
        padded, upper.astype(jnp.float32), (((1,), (0,)), ((), ())))

    bstart = excl * (1.0 / B)
    bi = jax.lax.broadcasted_iota(jnp.int32, (NB, E), 0).astype(jnp.float32)
    be = jnp.sum((bi >= bstart).astype(jnp.float32), axis=1, keepdims=True) - 1.0
    be_ref[...] = be.astype(jnp.int32)

    ltri = (jax.lax.broadcasted_iota(jnp.int32, (CH, CH), 0)
            > jax.lax.broadcasted_iota(jnp.int32, (CH, CH), 1)).astype(jnp.float32)
    carry = jnp.zeros((1, E), jnp.float32)
    chunks = []
    for c in range(T // CH):
        oh = onehot[c * CH:(c + 1) * CH]
        prefix = jax.lax.dot_general(ltri, oh, (((1,), (0,)), ((), ())))
        d = jnp.sum(oh * (prefix + carry + excl), axis=1, keepdims=True)
        chunks.append(d.astype(jnp.int32))
        carry = carry + jnp.sum(oh, axis=0, keepdims=True)
    destv = jnp.concatenate(chunks, axis=0)
    dest_ref[...] = destv

    ids_row = jax.lax.broadcasted_iota(jnp.int32, (1, T), 1).astype(jnp.float32)
    hi_row = jnp.floor(ids_row * (1.0 / 16.0))
    lo_row = (ids_row - 16.0 * hi_row).astype(jnp.bfloat16)
    hi_row = hi_row.astype(jnp.bfloat16)
    ones_row = jnp.ones((1, T), jnp.bfloat16)
    for c4 in range(PAD_T // SD):
        dio = jax.lax.broadcasted_iota(jnp.int32, (T, SD), 1) + c4 * SD
        hit = (destv == dio).astype(jnp.bfloat16)
        s_hi = jax.lax.dot_general(hi_row, hit, (((1,), (0,)), ((), ())),
                                   preferred_element_type=jnp.float32)
        s_lo = jax.lax.dot_general(lo_row, hit, (((1,), (0,)), ((), ())),
                                   preferred_element_type=jnp.float32)
        anyhit = jax.lax.dot_general(ones_row, hit, (((1,), (0,)), ((), ())),
                                     preferred_element_type=jnp.float32)
        fill = (jax.lax.broadcasted_iota(jnp.int32, (1, SD), 1)
                + (c4 * SD) % T).astype(jnp.float32)
        srt_c = 16.0 * s_hi + s_lo + (1.0 - anyhit) * fill
        srt_ref[c4:c4 + 1, :] = srt_c.astype(jnp.int32)


_route = pl.pallas_call(
    _route_body,
    out_shape=(jax.ShapeDtypeStruct((T, 1), jnp.int32),
               jax.ShapeDtypeStruct((NB, 1), jnp.int32),
               jax.ShapeDtypeStruct((PAD_T // SD, SD), jnp.int32)),
)


def _ffn_body(be_ref, xs_ref, w1_ref, w3_ref, w2_ref, out_ref):
    del be_ref
    xb = xs_ref[...].astype(jnp.bfloat16)
    w1 = w1_ref[0].astype(jnp.bfloat16)
    w3 = w3_ref[0].astype(jnp.bfloat16)
    w2 = w2_ref[0].astype(jnp.bfloat16)
    a = jax.lax.dot_general(xb, w1, (((1,), (1,)), ((), ())),
                            preferred_element_type=jnp.float32)
    g = jax.lax.dot_general(xb, w3, (((1,), (1,)), ((), ())),
                            preferred_element_type=jnp.float32)
    h = (a * jax.nn.sigmoid(a) * g).astype(jnp.bfloat16)
    out_ref[...] = jax.lax.dot_general(
        h, w2, (((1,), (1,)), ((), ())),
        preferred_element_type=jnp.float32)


_ffn = pl.pallas_call(
    _ffn_body,
    grid_spec=pltpu.PrefetchScalarGridSpec(
        num_scalar_prefetch=1,
        grid=(NB,),
        in_specs=[
            pl.BlockSpec((B, D), lambda i, be: (i, 0)),
            pl.BlockSpec((1, H, D), lambda i, be: (be[i], 0, 0)),
            pl.BlockSpec((1, H, D), lambda i, be: (be[i], 0, 0)),
            pl.BlockSpec((1, D, H), lambda i, be: (be[i], 0, 0)),
        ],
        out_specs=pl.BlockSpec((B, D), lambda i, be: (i, 0)),
    ),
    out_shape=jax.ShapeDtypeStruct((PAD_T, D), jnp.float32),
)


@functools.lru_cache(maxsize=1)
def _sc_kernels():
    mesh = plsc.VectorSubcoreMesh(core_axis_name="c", subcore_axis_name="s")
    @functools.partial(
        pl.kernel,
        mesh=mesh,
        out_type=jax.ShapeDtypeStruct((PAD_T, D), jnp.float32),
        scratch_types=[
            pltpu.VMEM((RPW2,), jnp.int32),
            pltpu.VMEM((RPW2, D), jnp.float32),
            pltpu.SemaphoreType.DMA,
        ],
    )
    def dispatch(x_hbm, srt_hbm, out_hbm, idx_v, rows_v, sem):
        wid = jax.lax.axis_index("s") * 2 + jax.lax.axis_index("c")
        base = wid * RPW2
        pltpu.sync_copy(srt_hbm.at[pl.ds(base, RPW2)], idx_v)
        pltpu.async_copy(x_hbm.at[idx_v], rows_v, sem).wait()
        pltpu.sync_copy(rows_v, out_hbm.at[pl.ds(base, RPW2)])

    @functools.partial(
        pl.kernel,
        mesh=mesh,
        out_type=jax.ShapeDtypeStruct((T, D), jnp.float32),
        scratch_types=[
            pltpu.VMEM((RPW,), jnp.int32),
            pltpu.VMEM((RPW, D), jnp.float32),
            pltpu.SemaphoreType.DMA,
        ],
    )
    def combine(ys_hbm, dest_hbm, out_hbm, idx_v, rows_v, sem):
        wid = jax.lax.axis_index("s") * 2 + jax.lax.axis_index("c")
        base = wid * RPW
        pltpu.sync_copy(dest_hbm.at[pl.ds(base, RPW)], idx_v)
        pltpu.async_copy(ys_hbm.at[idx_v], rows_v, sem).wait()
        pltpu.sync_copy(rows_v, out_hbm.at[pl.ds(base, RPW)])

    return dispatch, combine


def kernel(x, Wg, w1, w2, w3):
    orig_shape = x.shape
    xf = x.reshape(-1, x.shape[-1])
    scores = xf @ Wg.T
    _, expert_indices = jax.lax.top_k(scores, 1)
    dest2, be2, srt2 = _route(expert_indices)
    dest = dest2.reshape(T)
    be = be2.reshape(NB)
    srt = srt2.reshape(PAD_T)
    dispatch, combine = _sc_kernels()
    xs = dispatch(xf, srt)
    ys = _ffn(be, xs, w1, w3, w2)
    y = combine(ys, dest)
    return y.reshape(orig_shape)
